# baseline (speedup 1.0000x reference)
import functools

import numpy as np
import jax
import jax.numpy as jnp
from jax import lax
from jax.experimental import pallas as pl
from jax.experimental.pallas import tpu as pltpu
from jax.experimental.pallas import tpu_sc as plsc

N = 20000
E = 160000
IN_CH = 30
H = 16

NC = 2
NS = 16
NW = NC * NS
CHUNK = 128
ECH = E // CHUNK
BLK = 1600
NBLK = E // BLK

_DEF = jax.lax.Precision.DEFAULT

_EB1 = np.kron(np.eye(IN_CH, dtype=np.float32), np.ones((1, H), np.float32))
_ES1 = np.kron(np.ones((IN_CH, 1), np.float32), np.eye(H, dtype=np.float32))
_EB2 = np.kron(np.eye(H, dtype=np.float32), np.ones((1, H), np.float32))
_ES2 = np.kron(np.ones((H, 1), np.float32), np.eye(H, dtype=np.float32))


def _mesh():
    return plsc.VectorSubcoreMesh(
        core_axis_name="c", subcore_axis_name="s",
        num_cores=NC, num_subcores=NS)


def _exact_dot(a, b):
    f32, bf = jnp.float32, jnp.bfloat16
    t1 = a.astype(bf)
    r1 = a - t1.astype(f32)
    t2 = r1.astype(bf)
    r2 = r1 - t2.astype(f32)
    t3 = r2.astype(bf)
    b16 = b.astype(bf)
    return (jnp.dot(t1, b16, preferred_element_type=f32)
            + jnp.dot(t2, b16, preferred_element_type=f32)
            + jnp.dot(t3, b16, preferred_element_type=f32))


def _sc_gather(table, idx2d):
    R, W = table.shape
    CH = idx2d.shape[0]
    q, r = divmod(CH, NW)

    @functools.partial(
        pl.kernel,
        out_type=jax.ShapeDtypeStruct((CH * CHUNK, W), jnp.float32),
        mesh=_mesh(),
        scratch_types=[
            pltpu.VMEM((q + 1, CHUNK), jnp.int32),
            pltpu.VMEM((CHUNK, W), jnp.float32),
            pltpu.SemaphoreType.DMA,
        ],
        compiler_params=pltpu.CompilerParams(use_tc_tiling_on_sc=False),
    )
    def k(table_hbm, idx_hbm, out_hbm, idx_v, rows_v, sem):
        wid = lax.axis_index("s") * NC + lax.axis_index("c")
        base = wid * q
        pltpu.sync_copy(idx_hbm.at[pl.ds(base, q)], idx_v.at[pl.ds(0, q)])
        if r:
            @pl.when(wid < r)
            def _():
                pltpu.sync_copy(idx_hbm.at[pl.ds(NW * q + wid, 1)],
                                idx_v.at[pl.ds(q, 1)])

        def body(j, carry):
            pltpu.async_copy(table_hbm.at[idx_v.at[j]], rows_v, sem).wait()
            pltpu.sync_copy(rows_v, out_hbm.at[pl.ds((base + j) * CHUNK, CHUNK)])
            return carry

        lax.fori_loop(0, q, body, 0, unroll=False)
        if r:
            @pl.when(wid < r)
            def _():
                pltpu.async_copy(table_hbm.at[idx_v.at[q]], rows_v, sem).wait()
                pltpu.sync_copy(
                    rows_v, out_hbm.at[pl.ds((NW * q + wid) * CHUNK, CHUNK)])

    return k(table, idx2d)


def _sc_scatter_add(msg, dst2d):
    CH = dst2d.shape[0]
    q, r = divmod(CH, NW)
    rows_per_tile = N // NS

    @functools.partial(
        pl.kernel,
        out_type=jax.ShapeDtypeStruct((NC, N, H), jnp.float32),
        mesh=_mesh(),
        scratch_types=[
            pltpu.VMEM((q + 1, CHUNK), jnp.int32),
            pltpu.VMEM((CHUNK, H), jnp.float32),
            pltpu.VMEM((rows_per_tile, H), jnp.float32),
            pltpu.VMEM_SHARED((N, H), jnp.float32),
        ],
        compiler_params=pltpu.CompilerParams(use_tc_tiling_on_sc=False),
    )
    def k(msg_hbm, dst_hbm, agg_hbm, idx_v, mrow_v, zbuf_v, agg_sh):
        c = lax.axis_index("c")
        s = lax.axis_index("s")
        wid = s * NC + c

        def zb(i, carry):
            zbuf_v[i, :] = jnp.zeros((H,), jnp.float32)
            return carry
        lax.fori_loop(0, rows_per_tile, zb, 0)
        pltpu.sync_copy(zbuf_v, agg_sh.at[pl.ds(s * rows_per_tile, rows_per_tile)])
        plsc.subcore_barrier()

        base = wid * q
        pltpu.sync_copy(dst_hbm.at[pl.ds(base, q)], idx_v.at[pl.ds(0, q)])
        if r:
            @pl.when(wid < r)
            def _():
                pltpu.sync_copy(dst_hbm.at[pl.ds(NW * q + wid, 1)],
                                idx_v.at[pl.ds(q, 1)])

        def body(j, carry):
            pltpu.sync_copy(msg_hbm.at[pl.ds((base + j) * CHUNK, CHUNK)], mrow_v)
            pltpu.sync_copy(mrow_v, agg_sh.at[idx_v.at[j]], add=True)
            return carry
        lax.fori_loop(0, q, body, 0, unroll=False)
        if r:
            @pl.when(wid < r)
            def _():
                pltpu.sync_copy(
                    msg_hbm.at[pl.ds((NW * q + wid) * CHUNK, CHUNK)], mrow_v)
                pltpu.sync_copy(mrow_v, agg_sh.at[idx_v.at[q]], add=True)
        plsc.subcore_barrier()

        pltpu.sync_copy(
            agg_sh.at[pl.ds(s * rows_per_tile, rows_per_tile)],
            agg_hbm.at[c].at[pl.ds(s * rows_per_tile, rows_per_tile)])

    return k(msg, dst2d)


def _tc_msg(xsd, feat_src, w_in, b_in, eb, es, in_ch):
    K = in_ch * H

    def body(xs_ref, xd_ref, f_ref, w_ref, b_ref, eb_ref, es_ref, out_ref):
        rel = xs_ref[:, 0:2] - xd_ref[:, 0:2]
        scaling = jnp.dot(rel.astype(jnp.bfloat16),
                          w_ref[...].astype(jnp.bfloat16),
                          preferred_element_type=jnp.float32)
        scaling = jnp.maximum(scaling + b_ref[...], 0.0)
        if f_ref.shape[1] == 32:
            feat = f_ref[:, 2:32]
        else:
            feat = f_ref[...]
        acc = scaling[:, 0:H] * feat[:, 0:1]
        for c in range(1, in_ch):
            acc = acc + scaling[:, c * H:(c + 1) * H] * feat[:, c:c + 1]
        out_ref[...] = acc

    fw = feat_src.shape[1]
    return pl.pallas_call(
        body,
        grid=(NBLK,),
        in_specs=[
            pl.BlockSpec((BLK, 32), lambda i: (i, 0)),
            pl.BlockSpec((BLK, 32), lambda i: (i + NBLK, 0)),
            pl.BlockSpec((BLK, fw), lambda i: (i, 0)),
            pl.BlockSpec((2, K), lambda i: (0, 0)),
            pl.BlockSpec((1, K), lambda i: (0, 0)),
            pl.BlockSpec((in_ch, K), lambda i: (0, 0)),
            pl.BlockSpec((K, H), lambda i: (0, 0)),
        ],
        out_specs=pl.BlockSpec((BLK, H), lambda i: (i, 0)),
        out_shape=jax.ShapeDtypeStruct((E, H), jnp.float32),
    )(xsd, xsd, feat_src, w_in, b_in.reshape(1, K), eb, es)


def _tc_update(agg, w_out, b_out):
    RB = 4000
    def body(a0_ref, a1_ref, w_ref, b_ref, out_ref):
        a = (a0_ref[0] + a1_ref[0]).astype(jnp.bfloat16)
        out_ref[...] = jnp.dot(a, w_ref[...].astype(jnp.bfloat16),
                               preferred_element_type=jnp.float32) + b_ref[...]

    return pl.pallas_call(
        body,
        grid=(N // RB,),
        in_specs=[
            pl.BlockSpec((1, RB, H), lambda i: (0, i, 0)),
            pl.BlockSpec((1, RB, H), lambda i: (1, i, 0)),
            pl.BlockSpec((H, H), lambda i: (0, 0)),
            pl.BlockSpec((1, H), lambda i: (0, 0)),
        ],
        out_specs=pl.BlockSpec((RB, H), lambda i: (i, 0)),
        out_shape=jax.ShapeDtypeStruct((N, H), jnp.float32),
    )(agg, agg, w_out, b_out.reshape(1, H))


def _tc_decoder(hsd, dw1, db1, dw2, db2, dw3, db3):
    DB = 3200
    ND = E // DB

    def body(hs_ref, hd_ref, w1a_ref, w1b_ref, b1_ref, w2_ref, b2_ref,
             w3_ref, b3_ref, out_ref):
        f32 = jnp.float32
        bf = jnp.bfloat16
        z = (jnp.dot(hs_ref[...].astype(bf), w1a_ref[...].astype(bf),
                     preferred_element_type=f32)
             + jnp.dot(hd_ref[...].astype(bf), w1b_ref[...].astype(bf),
                       preferred_element_type=f32)
             + b1_ref[...])
        z = jnp.maximum(z, 0.0)
        z = jnp.maximum(jnp.dot(z.astype(bf), w2_ref[...].astype(bf),
                                preferred_element_type=f32) + b2_ref[...], 0.0)
        z = jnp.dot(z.astype(bf), w3_ref[...].astype(bf),
                    preferred_element_type=f32) + b3_ref[...]
        out_ref[...] = jax.nn.sigmoid(z)

    return pl.pallas_call(
        body,
        grid=(ND,),
        in_specs=[
            pl.BlockSpec((DB, H), lambda i: (i, 0)),
            pl.BlockSpec((DB, H), lambda i: (i + ND, 0)),
            pl.BlockSpec((H, H), lambda i: (0, 0)),
            pl.BlockSpec((H, H), lambda i: (0, 0)),
            pl.BlockSpec((1, H), lambda i: (0, 0)),
            pl.BlockSpec((H, H), lambda i: (0, 0)),
            pl.BlockSpec((1, H), lambda i: (0, 0)),
            pl.BlockSpec((H, 1), lambda i: (0, 0)),
            pl.BlockSpec((1, 1), lambda i: (0, 0)),
        ],
        out_specs=pl.BlockSpec((DB, 1), lambda i: (i, 0)),
        out_shape=jax.ShapeDtypeStruct((E, 1), jnp.float32),
    )(hsd, hsd, dw1[:H], dw1[H:], db1.reshape(1, H), dw2, db2.reshape(1, H),
      dw3, db3.reshape(1, 1))


def kernel(x, edge_index, w1_in, b1_in, w1_out, b1_out, w2_in, b2_in, w2_out,
           b2_out, w3_in, b3_in, w3_out, b3_out, dw1, db1, dw2, db2, dw3, db3):
    ei = edge_index.astype(jnp.int32)
    idx_sd = ei.reshape(2 * ECH, CHUNK)
    idx_s = ei[0].reshape(ECH, CHUNK)
    idx_d = ei[1].reshape(ECH, CHUNK)

    eb1, es1 = jnp.asarray(_EB1), jnp.asarray(_ES1)
    eb2, es2 = jnp.asarray(_EB2), jnp.asarray(_ES2)

    xsd = _sc_gather(x, idx_sd)
    msg1 = _tc_msg(xsd, xsd, w1_in, b1_in, eb1, es1, IN_CH)
    agg1 = _sc_scatter_add(msg1, idx_d)
    h1 = _tc_update(agg1, w1_out, b1_out)
    hs1 = _sc_gather(h1, idx_s)
    msg2 = _tc_msg(xsd, hs1, w2_in, b2_in, eb2, es2, H)
    agg2 = _sc_scatter_add(msg2, idx_d)
    h2 = _tc_update(agg2, w2_out, b2_out)
    hs2 = _sc_gather(h2, idx_s)
    msg3 = _tc_msg(xsd, hs2, w3_in, b3_in, eb2, es2, H)
    agg3 = _sc_scatter_add(msg3, idx_d)
    h3 = _tc_update(agg3, w3_out, b3_out)
    hsd3 = _sc_gather(h3, idx_sd)
    z = _tc_decoder(hsd3, dw1, db1, dw2, db2, dw3, db3)
    return z.reshape(-1)


# speedup vs baseline: 1.7803x; 1.7803x over previous
import functools

import numpy as np
import jax
import jax.numpy as jnp
from jax import lax
from jax.experimental import pallas as pl
from jax.experimental.pallas import tpu as pltpu
from jax.experimental.pallas import tpu_sc as plsc

N = 20000
E = 160000
IN_CH = 30
H = 16

NC = 2
NS = 16
NW = NC * NS
CHUNK = 128
ECH = E // CHUNK
BLK = 1600
NBLK = E // BLK

_DEF = jax.lax.Precision.DEFAULT

_EB1 = np.kron(np.eye(IN_CH, dtype=np.float32), np.ones((1, H), np.float32))
_ES1 = np.kron(np.ones((IN_CH, 1), np.float32), np.eye(H, dtype=np.float32))
_EB2 = np.kron(np.eye(H, dtype=np.float32), np.ones((1, H), np.float32))
_ES2 = np.kron(np.ones((H, 1), np.float32), np.eye(H, dtype=np.float32))


def _mesh():
    return plsc.VectorSubcoreMesh(
        core_axis_name="c", subcore_axis_name="s",
        num_cores=NC, num_subcores=NS)


def _exact_dot(a, b):
    f32, bf = jnp.float32, jnp.bfloat16
    t1 = a.astype(bf)
    r1 = a - t1.astype(f32)
    t2 = r1.astype(bf)
    r2 = r1 - t2.astype(f32)
    t3 = r2.astype(bf)
    b16 = b.astype(bf)
    return (jnp.dot(t1, b16, preferred_element_type=f32)
            + jnp.dot(t2, b16, preferred_element_type=f32)
            + jnp.dot(t3, b16, preferred_element_type=f32))


def _sc_gather(table, idx2d):
    R, W = table.shape
    CH = idx2d.shape[0]
    q, r = divmod(CH, NW)

    @functools.partial(
        pl.kernel,
        out_type=jax.ShapeDtypeStruct((CH * CHUNK, W), jnp.float32),
        mesh=_mesh(),
        scratch_types=[
            pltpu.VMEM((q + 1, CHUNK), jnp.int32),
            pltpu.VMEM((CHUNK, W), jnp.float32),
            pltpu.SemaphoreType.DMA,
        ],
        compiler_params=pltpu.CompilerParams(use_tc_tiling_on_sc=False),
    )
    def k(table_hbm, idx_hbm, out_hbm, idx_v, rows_v, sem):
        wid = lax.axis_index("s") * NC + lax.axis_index("c")
        base = wid * q
        pltpu.sync_copy(idx_hbm.at[pl.ds(base, q)], idx_v.at[pl.ds(0, q)])
        if r:
            @pl.when(wid < r)
            def _():
                pltpu.sync_copy(idx_hbm.at[pl.ds(NW * q + wid, 1)],
                                idx_v.at[pl.ds(q, 1)])

        def body(j, carry):
            pltpu.async_copy(table_hbm.at[idx_v.at[j]], rows_v, sem).wait()
            pltpu.sync_copy(rows_v, out_hbm.at[pl.ds((base + j) * CHUNK, CHUNK)])
            return carry

        lax.fori_loop(0, q, body, 0, unroll=False)
        if r:
            @pl.when(wid < r)
            def _():
                pltpu.async_copy(table_hbm.at[idx_v.at[q]], rows_v, sem).wait()
                pltpu.sync_copy(
                    rows_v, out_hbm.at[pl.ds((NW * q + wid) * CHUNK, CHUNK)])

    return k(table, idx2d)


def _sc_scatter_add(msg, dst2d):
    CH = dst2d.shape[0]
    q, r = divmod(CH, NW)
    rows_per_tile = N // NS

    @functools.partial(
        pl.kernel,
        out_type=jax.ShapeDtypeStruct((NC, N, H), jnp.float32),
        mesh=_mesh(),
        scratch_types=[
            pltpu.VMEM((q + 1, CHUNK), jnp.int32),
            pltpu.VMEM((CHUNK, H), jnp.float32),
            pltpu.VMEM((rows_per_tile, H), jnp.float32),
            pltpu.VMEM_SHARED((N, H), jnp.float32),
        ],
        compiler_params=pltpu.CompilerParams(use_tc_tiling_on_sc=False),
    )
    def k(msg_hbm, dst_hbm, agg_hbm, idx_v, mrow_v, zbuf_v, agg_sh):
        c = lax.axis_index("c")
        s = lax.axis_index("s")
        wid = s * NC + c

        def zb(i, carry):
            zbuf_v[i, :] = jnp.zeros((H,), jnp.float32)
            return carry
        lax.fori_loop(0, rows_per_tile, zb, 0)
        pltpu.sync_copy(zbuf_v, agg_sh.at[pl.ds(s * rows_per_tile, rows_per_tile)])
        plsc.subcore_barrier()

        base = wid * q
        pltpu.sync_copy(dst_hbm.at[pl.ds(base, q)], idx_v.at[pl.ds(0, q)])
        if r:
            @pl.when(wid < r)
            def _():
                pltpu.sync_copy(dst_hbm.at[pl.ds(NW * q + wid, 1)],
                                idx_v.at[pl.ds(q, 1)])

        def body(j, carry):
            pltpu.sync_copy(msg_hbm.at[pl.ds((base + j) * CHUNK, CHUNK)], mrow_v)
            pltpu.sync_copy(mrow_v, agg_sh.at[idx_v.at[j]], add=True)
            return carry
        lax.fori_loop(0, q, body, 0, unroll=False)
        if r:
            @pl.when(wid < r)
            def _():
                pltpu.sync_copy(
                    msg_hbm.at[pl.ds((NW * q + wid) * CHUNK, CHUNK)], mrow_v)
                pltpu.sync_copy(mrow_v, agg_sh.at[idx_v.at[q]], add=True)
        plsc.subcore_barrier()

        pltpu.sync_copy(
            agg_sh.at[pl.ds(s * rows_per_tile, rows_per_tile)],
            agg_hbm.at[c].at[pl.ds(s * rows_per_tile, rows_per_tile)])

    return k(msg, dst2d)


def _tc_msg(xsd, feat_src, w_in, b_in, eb, es, in_ch):
    K = in_ch * H

    def body(xs_ref, xd_ref, f_ref, w_ref, b_ref, eb_ref, es_ref, out_ref):
        rel = xs_ref[:, 0:2] - xd_ref[:, 0:2]
        scaling = jnp.dot(rel.astype(jnp.bfloat16),
                          w_ref[...].astype(jnp.bfloat16),
                          preferred_element_type=jnp.float32)
        scaling = jnp.maximum(scaling + b_ref[...], 0.0)
        if f_ref.shape[1] == 32:
            feat = f_ref[:, 2:32]
        else:
            feat = f_ref[...]
        frep = _exact_dot(feat, eb_ref[...])
        out_ref[...] = _exact_dot(scaling * frep, es_ref[...])

    fw = feat_src.shape[1]
    return pl.pallas_call(
        body,
        grid=(NBLK,),
        in_specs=[
            pl.BlockSpec((BLK, 32), lambda i: (i, 0)),
            pl.BlockSpec((BLK, 32), lambda i: (i + NBLK, 0)),
            pl.BlockSpec((BLK, fw), lambda i: (i, 0)),
            pl.BlockSpec((2, K), lambda i: (0, 0)),
            pl.BlockSpec((1, K), lambda i: (0, 0)),
            pl.BlockSpec((in_ch, K), lambda i: (0, 0)),
            pl.BlockSpec((K, H), lambda i: (0, 0)),
        ],
        out_specs=pl.BlockSpec((BLK, H), lambda i: (i, 0)),
        out_shape=jax.ShapeDtypeStruct((E, H), jnp.float32),
    )(xsd, xsd, feat_src, w_in, b_in.reshape(1, K), eb, es)


def _tc_update(agg, w_out, b_out):
    RB = 4000
    def body(a0_ref, a1_ref, w_ref, b_ref, out_ref):
        a = (a0_ref[0] + a1_ref[0]).astype(jnp.bfloat16)
        out_ref[...] = jnp.dot(a, w_ref[...].astype(jnp.bfloat16),
                               preferred_element_type=jnp.float32) + b_ref[...]

    return pl.pallas_call(
        body,
        grid=(N // RB,),
        in_specs=[
            pl.BlockSpec((1, RB, H), lambda i: (0, i, 0)),
            pl.BlockSpec((1, RB, H), lambda i: (1, i, 0)),
            pl.BlockSpec((H, H), lambda i: (0, 0)),
            pl.BlockSpec((1, H), lambda i: (0, 0)),
        ],
        out_specs=pl.BlockSpec((RB, H), lambda i: (i, 0)),
        out_shape=jax.ShapeDtypeStruct((N, H), jnp.float32),
    )(agg, agg, w_out, b_out.reshape(1, H))


def _tc_decoder(hsd, dw1, db1, dw2, db2, dw3, db3):
    DB = 3200
    ND = E // DB

    def body(hs_ref, hd_ref, w1a_ref, w1b_ref, b1_ref, w2_ref, b2_ref,
             w3_ref, b3_ref, out_ref):
        f32 = jnp.float32
        bf = jnp.bfloat16
        z = (jnp.dot(hs_ref[...].astype(bf), w1a_ref[...].astype(bf),
                     preferred_element_type=f32)
             + jnp.dot(hd_ref[...].astype(bf), w1b_ref[...].astype(bf),
                       preferred_element_type=f32)
             + b1_ref[...])
        z = jnp.maximum(z, 0.0)
        z = jnp.maximum(jnp.dot(z.astype(bf), w2_ref[...].astype(bf),
                                preferred_element_type=f32) + b2_ref[...], 0.0)
        z = jnp.dot(z.astype(bf), w3_ref[...].astype(bf),
                    preferred_element_type=f32) + b3_ref[...]
        out_ref[...] = jax.nn.sigmoid(z)

    return pl.pallas_call(
        body,
        grid=(ND,),
        in_specs=[
            pl.BlockSpec((DB, H), lambda i: (i, 0)),
            pl.BlockSpec((DB, H), lambda i: (i + ND, 0)),
            pl.BlockSpec((H, H), lambda i: (0, 0)),
            pl.BlockSpec((H, H), lambda i: (0, 0)),
            pl.BlockSpec((1, H), lambda i: (0, 0)),
            pl.BlockSpec((H, H), lambda i: (0, 0)),
            pl.BlockSpec((1, H), lambda i: (0, 0)),
            pl.BlockSpec((H, 1), lambda i: (0, 0)),
            pl.BlockSpec((1, 1), lambda i: (0, 0)),
        ],
        out_specs=pl.BlockSpec((DB, 1), lambda i: (i, 0)),
        out_shape=jax.ShapeDtypeStruct((E, 1), jnp.float32),
    )(hsd, hsd, dw1[:H], dw1[H:], db1.reshape(1, H), dw2, db2.reshape(1, H),
      dw3, db3.reshape(1, 1))


def kernel(x, edge_index, w1_in, b1_in, w1_out, b1_out, w2_in, b2_in, w2_out,
           b2_out, w3_in, b3_in, w3_out, b3_out, dw1, db1, dw2, db2, dw3, db3):
    ei = edge_index.astype(jnp.int32)
    idx_sd = ei.reshape(2 * ECH, CHUNK)
    idx_s = ei[0].reshape(ECH, CHUNK)
    idx_d = ei[1].reshape(ECH, CHUNK)

    eb1, es1 = jnp.asarray(_EB1), jnp.asarray(_ES1)
    eb2, es2 = jnp.asarray(_EB2), jnp.asarray(_ES2)

    xsd = _sc_gather(x, idx_sd)
    msg1 = _tc_msg(xsd, xsd, w1_in, b1_in, eb1, es1, IN_CH)
    agg1 = _sc_scatter_add(msg1, idx_d)
    h1 = _tc_update(agg1, w1_out, b1_out)
    hs1 = _sc_gather(h1, idx_s)
    msg2 = _tc_msg(xsd, hs1, w2_in, b2_in, eb2, es2, H)
    agg2 = _sc_scatter_add(msg2, idx_d)
    h2 = _tc_update(agg2, w2_out, b2_out)
    hs2 = _sc_gather(h2, idx_s)
    msg3 = _tc_msg(xsd, hs2, w3_in, b3_in, eb2, es2, H)
    agg3 = _sc_scatter_add(msg3, idx_d)
    h3 = _tc_update(agg3, w3_out, b3_out)
    hsd3 = _sc_gather(h3, idx_sd)
    z = _tc_decoder(hsd3, dw1, db1, dw2, db2, dw3, db3)
    return z.reshape(-1)
